# baseline (speedup 1.0000x reference)
import functools

import jax
import jax.numpy as jnp
from jax.experimental import pallas as pl

RADIUS = 4
TEMP = 2.0
NEG_FILL = jnp.finfo(jnp.float32).min


def _tile_kernel(x_ref, w_ref, ss_ref, sel_ref, out_ref, *, n_states):
    x = x_ref[...]
    w = w_ref[...]
    B, DBLK = x.shape
    LBLK = DBLK * n_states
    lo = ss_ref[0, 0]
    ls = ss_ref[0, 1] - ss_ref[0, 0]

    s = jnp.round((x - lo) / ls)
    u = (w * x) * (-ls / TEMP)
    u_hi = u.astype(jnp.bfloat16)
    u_lo = (u - u_hi.astype(jnp.float32)).astype(jnp.bfloat16)
    stack = jnp.concatenate(
        [s.astype(jnp.bfloat16), u_hi, u_lo], axis=0)
    rep = jnp.dot(stack, sel_ref[...],
                  preferred_element_type=jnp.float32)
    s_r = rep[:B]
    u_r = rep[B:2 * B] + rep[2 * B:]

    jf = jax.lax.broadcasted_iota(jnp.int32, (1, LBLK), 1) % n_states
    delta = jf.astype(jnp.float32) - s_r
    adelta = jnp.abs(delta)
    mask = (adelta >= 1.0) & (adelta <= float(RADIUS))
    out_ref[...] = jnp.where(mask, u_r * delta, NEG_FILL)


def kernel(x, w, state_space):
    B, D = x.shape
    NS = state_space.shape[0]
    DBLK = 512
    LBLK = DBLK * NS
    sel = (jnp.arange(LBLK, dtype=jnp.int32)[None, :] // NS
           == jnp.arange(DBLK, dtype=jnp.int32)[:, None]).astype(jnp.bfloat16)
    grid = (D // DBLK,)
    out = pl.pallas_call(
        functools.partial(_tile_kernel, n_states=NS),
        grid=grid,
        in_specs=[
            pl.BlockSpec((B, DBLK), lambda i: (0, i)),
            pl.BlockSpec((1, DBLK), lambda i: (0, i)),
            pl.BlockSpec((1, NS), lambda i: (0, 0)),
            pl.BlockSpec((DBLK, LBLK), lambda i: (0, 0)),
        ],
        out_specs=pl.BlockSpec((B, LBLK), lambda i: (0, i)),
        out_shape=jax.ShapeDtypeStruct((B, D * NS), jnp.float32),
    )(x, w.reshape(1, D), state_space.reshape(1, NS), sel)
    return out
